# Initial kernel scaffold; baseline (speedup 1.0000x reference)
import jax
import jax.numpy as jnp
from jax.experimental import pallas as pl


def kernel(inputs, token_table, pos_table, gamma, beta):
    raise NotImplementedError("write your pallas kernel here")


# speedup vs baseline: 1.5318x; 1.5318x over previous
import functools

import jax
import jax.numpy as jnp
from jax import lax
from jax.experimental import pallas as pl
from jax.experimental.pallas import tpu as pltpu
from jax.experimental.pallas import tpu_sc as plsc

_D_MODEL = 128
_SEQ = 512
_SCALE = 1
_EPS = 1e-6

_NC = 2
_NS = 16
_NW = _NC * _NS

_CH = 128


def _sc_gather(table, flat_idx):
    n = flat_idx.shape[0]
    b_per_w = n // _NW
    n_chunks = b_per_w // _CH
    mesh = plsc.VectorSubcoreMesh(core_axis_name="c", subcore_axis_name="s")

    @functools.partial(
        pl.kernel,
        out_type=jax.ShapeDtypeStruct((n, _D_MODEL), jnp.float32),
        mesh=mesh,
        scratch_types=[
            pltpu.VMEM((b_per_w,), jnp.int32),
            pltpu.VMEM((_CH, _D_MODEL), jnp.float32),
        ],
    )
    def gather_kernel(table_hbm, idx_hbm, out_hbm, idx_v, buf):
        wid = lax.axis_index("s") * _NC + lax.axis_index("c")
        base = wid * b_per_w
        pltpu.sync_copy(idx_hbm.at[pl.ds(base, b_per_w)], idx_v)

        @pl.loop(0, n_chunks)
        def _(c):
            off = c * _CH
            pltpu.sync_copy(table_hbm.at[idx_v.at[pl.ds(off, _CH)]], buf)
            pltpu.sync_copy(buf, out_hbm.at[pl.ds(base + off, _CH)])

    return gather_kernel(table, flat_idx)


def _ln_body(x_ref, pos_ref, g_ref, b_ref, o_ref):
    x = x_ref[...] * _SCALE + pos_ref[...]
    mean = jnp.mean(x, axis=-1, keepdims=True)
    xc = x - mean
    var = jnp.mean(xc * xc, axis=-1, keepdims=True)
    y = xc * lax.rsqrt(var + _EPS)
    o_ref[...] = y * g_ref[...] + b_ref[...]


def _tc_layernorm(gathered, pos_table, gamma, beta):
    batch = gathered.shape[0]
    bb = 8
    grid = (batch // bb,)
    return pl.pallas_call(
        _ln_body,
        grid=grid,
        in_specs=[
            pl.BlockSpec((bb, _SEQ, _D_MODEL), lambda i: (i, 0, 0)),
            pl.BlockSpec((_SEQ, _D_MODEL), lambda i: (0, 0)),
            pl.BlockSpec((1, 1, _D_MODEL), lambda i: (0, 0, 0)),
            pl.BlockSpec((1, 1, _D_MODEL), lambda i: (0, 0, 0)),
        ],
        out_specs=pl.BlockSpec((bb, _SEQ, _D_MODEL), lambda i: (i, 0, 0)),
        out_shape=jax.ShapeDtypeStruct((batch, _SEQ, _D_MODEL), jnp.float32),
    )(gathered, pos_table, gamma, beta)


@jax.jit
def kernel(inputs, token_table, pos_table, gamma, beta):
    batch, seq = inputs.shape
    flat_idx = inputs.reshape(-1)
    gathered = _sc_gather(token_table, flat_idx)
    gathered = gathered.reshape(batch, seq, _D_MODEL)
    return _tc_layernorm(
        gathered,
        pos_table,
        gamma.reshape(1, 1, _D_MODEL),
        beta.reshape(1, 1, _D_MODEL),
    )


# speedup vs baseline: 1.8423x; 1.2027x over previous
import functools

import jax
import jax.numpy as jnp
from jax import lax
from jax.experimental import pallas as pl
from jax.experimental.pallas import tpu as pltpu
from jax.experimental.pallas import tpu_sc as plsc

_D_MODEL = 128
_SEQ = 512
_SCALE = 1
_EPS = 1e-6

_NC = 2
_NS = 16
_NW = _NC * _NS

_CH = 128


def _sc_gather(table, flat_idx):
    n = flat_idx.shape[0]
    b_per_w = n // _NW
    n_chunks = b_per_w // _CH
    mesh = plsc.VectorSubcoreMesh(core_axis_name="c", subcore_axis_name="s")

    nbuf = 4

    @functools.partial(
        pl.kernel,
        out_type=jax.ShapeDtypeStruct((n, _D_MODEL), jnp.float32),
        mesh=mesh,
        scratch_types=[
            pltpu.VMEM((b_per_w,), jnp.int32),
            [pltpu.VMEM((_CH, _D_MODEL), jnp.float32) for _ in range(nbuf)],
            [pltpu.SemaphoreType.DMA for _ in range(nbuf)],
            [pltpu.SemaphoreType.DMA for _ in range(nbuf)],
        ],
    )
    def gather_kernel(table_hbm, idx_hbm, out_hbm, idx_v, bufs, gsems, wsems):
        wid = lax.axis_index("s") * _NC + lax.axis_index("c")
        base = wid * b_per_w
        pltpu.sync_copy(idx_hbm.at[pl.ds(base, b_per_w)], idx_v)

        def start_gather(c, b):
            pltpu.async_copy(
                table_hbm.at[idx_v.at[pl.ds(c * _CH, _CH)]], bufs[b], gsems[b]
            )

        def wait_gather(b):
            pltpu.make_async_copy(
                table_hbm.at[idx_v.at[pl.ds(0, _CH)]], bufs[b], gsems[b]
            ).wait()

        def start_write(c, b):
            pltpu.async_copy(
                bufs[b], out_hbm.at[pl.ds(base + c * _CH, _CH)], wsems[b]
            )

        def wait_write(b):
            pltpu.make_async_copy(
                bufs[b], out_hbm.at[pl.ds(base, _CH)], wsems[b]
            ).wait()

        for b in range(nbuf):
            start_gather(b, b)

        @pl.loop(0, n_chunks, step=nbuf)
        def _(c):
            for b in range(nbuf):
                wait_gather(b)
                start_write(c + b, b)

            for b in range(nbuf):
                @pl.when(c + nbuf + b < n_chunks)
                def _():
                    wait_write(b)
                    start_gather(c + nbuf + b, b)

        for b in range(nbuf):
            wait_write(b)

    return gather_kernel(table, flat_idx)


def _ln_body(x_ref, pos_ref, g_ref, b_ref, o_ref):
    x = x_ref[...] * _SCALE + pos_ref[...]
    mean = jnp.mean(x, axis=-1, keepdims=True)
    xc = x - mean
    var = jnp.mean(xc * xc, axis=-1, keepdims=True)
    y = xc * lax.rsqrt(var + _EPS)
    o_ref[...] = y * g_ref[...] + b_ref[...]


def _tc_layernorm(gathered, pos_table, gamma, beta):
    batch = gathered.shape[0]
    bb = 8
    grid = (batch // bb,)
    return pl.pallas_call(
        _ln_body,
        grid=grid,
        in_specs=[
            pl.BlockSpec((bb, _SEQ, _D_MODEL), lambda i: (i, 0, 0)),
            pl.BlockSpec((_SEQ, _D_MODEL), lambda i: (0, 0)),
            pl.BlockSpec((1, 1, _D_MODEL), lambda i: (0, 0, 0)),
            pl.BlockSpec((1, 1, _D_MODEL), lambda i: (0, 0, 0)),
        ],
        out_specs=pl.BlockSpec((bb, _SEQ, _D_MODEL), lambda i: (i, 0, 0)),
        out_shape=jax.ShapeDtypeStruct((batch, _SEQ, _D_MODEL), jnp.float32),
    )(gathered, pos_table, gamma, beta)


@jax.jit
def kernel(inputs, token_table, pos_table, gamma, beta):
    batch, seq = inputs.shape
    flat_idx = inputs.reshape(-1)
    gathered = _sc_gather(token_table, flat_idx)
    gathered = gathered.reshape(batch, seq, _D_MODEL)
    return _tc_layernorm(
        gathered,
        pos_table,
        gamma.reshape(1, 1, _D_MODEL),
        beta.reshape(1, 1, _D_MODEL),
    )


# speedup vs baseline: 2.0230x; 1.0981x over previous
import functools

import jax
import jax.numpy as jnp
from jax import lax
from jax.experimental import pallas as pl
from jax.experimental.pallas import tpu as pltpu
from jax.experimental.pallas import tpu_sc as plsc

_D_MODEL = 128
_SEQ = 512
_SCALE = 1
_EPS = 1e-6

_NC = 2
_NS = 16
_NW = _NC * _NS

_CH = 128


def _sc_gather(table, flat_idx):
    n = flat_idx.shape[0]
    b_per_w = n // _NW
    n_chunks = b_per_w // _CH
    mesh = plsc.VectorSubcoreMesh(core_axis_name="c", subcore_axis_name="s")

    nbuf = 4

    @functools.partial(
        pl.kernel,
        out_type=jax.ShapeDtypeStruct((n, _D_MODEL), jnp.float32),
        mesh=mesh,
        scratch_types=[
            pltpu.VMEM((b_per_w,), jnp.int32),
            [pltpu.VMEM((_CH, _D_MODEL), jnp.float32) for _ in range(nbuf)],
            [pltpu.SemaphoreType.DMA for _ in range(nbuf)],
            [pltpu.SemaphoreType.DMA for _ in range(nbuf)],
        ],
    )
    def gather_kernel(table_hbm, idx_hbm, out_hbm, idx_v, bufs, gsems, wsems):
        wid = lax.axis_index("s") * _NC + lax.axis_index("c")
        base = wid * b_per_w
        pltpu.sync_copy(idx_hbm.at[pl.ds(base, b_per_w)], idx_v)

        def start_gather(c, b):
            pltpu.async_copy(
                table_hbm.at[idx_v.at[pl.ds(c * _CH, _CH)]], bufs[b], gsems[b]
            )

        def wait_gather(b):
            pltpu.make_async_copy(
                table_hbm.at[idx_v.at[pl.ds(0, _CH)]], bufs[b], gsems[b]
            ).wait()

        def start_write(c, b):
            pltpu.async_copy(
                bufs[b], out_hbm.at[pl.ds(base + c * _CH, _CH)], wsems[b]
            )

        def wait_write(b):
            pltpu.make_async_copy(
                bufs[b], out_hbm.at[pl.ds(base, _CH)], wsems[b]
            ).wait()

        for b in range(nbuf):
            start_gather(b, b)

        @pl.loop(0, n_chunks, step=nbuf)
        def _(c):
            for b in range(nbuf):
                wait_gather(b)
                start_write(c + b, b)

            for b in range(nbuf):
                @pl.when(c + nbuf + b < n_chunks)
                def _():
                    wait_write(b)
                    start_gather(c + nbuf + b, b)

        for b in range(nbuf):
            wait_write(b)

    return gather_kernel(table, flat_idx)


def _ln_body_first(x_ref, pos_ref, g_ref, b_ref, o_ref):
    x = x_ref[...] * _SCALE + pos_ref[...]
    mean = jnp.mean(x, axis=-1, keepdims=True)
    xc = x - mean
    var = jnp.mean(xc * xc, axis=-1, keepdims=True)
    y = xc * lax.rsqrt(var + _EPS)
    o_ref[...] = y * g_ref[...] + b_ref[...]


def _ln_body_chained(acc_ref, x_ref, pos_ref, g_ref, b_ref, o_ref):
    del acc_ref
    _ln_body_first(x_ref, pos_ref, g_ref, b_ref, o_ref)


def _tc_layernorm_slice(acc, gathered, pos_table, gamma, beta, blk_off, batch):
    sl_batch = gathered.shape[0]
    bb = 8
    grid = (sl_batch // bb,)
    x_spec = pl.BlockSpec((bb, _SEQ, _D_MODEL), lambda i: (i, 0, 0))
    common_specs = [
        x_spec,
        pl.BlockSpec((_SEQ, _D_MODEL), lambda i: (0, 0)),
        pl.BlockSpec((1, 1, _D_MODEL), lambda i: (0, 0, 0)),
        pl.BlockSpec((1, 1, _D_MODEL), lambda i: (0, 0, 0)),
    ]
    out_spec = pl.BlockSpec(
        (bb, _SEQ, _D_MODEL), lambda i: (blk_off + i, 0, 0)
    )
    out_shape = jax.ShapeDtypeStruct((batch, _SEQ, _D_MODEL), jnp.float32)
    if acc is None:
        return pl.pallas_call(
            _ln_body_first,
            grid=grid,
            in_specs=common_specs,
            out_specs=out_spec,
            out_shape=out_shape,
        )(gathered, pos_table, gamma, beta)
    return pl.pallas_call(
        _ln_body_chained,
        grid=grid,
        in_specs=[pl.BlockSpec(memory_space=pl.ANY)] + common_specs,
        out_specs=out_spec,
        out_shape=out_shape,
        input_output_aliases={0: 0},
    )(acc, gathered, pos_table, gamma, beta)


_N_SLICES = 4


@jax.jit
def kernel(inputs, token_table, pos_table, gamma, beta):
    batch, seq = inputs.shape
    flat_idx = inputs.reshape(-1)
    gamma3 = gamma.reshape(1, 1, _D_MODEL)
    beta3 = beta.reshape(1, 1, _D_MODEL)
    sl_batch = batch // _N_SLICES
    sl_n = sl_batch * seq
    bb = 8

    gathered = [
        _sc_gather(token_table, lax.dynamic_slice(flat_idx, (s * sl_n,), (sl_n,)))
        for s in range(_N_SLICES)
    ]
    acc = None
    for s in range(_N_SLICES):
        acc = _tc_layernorm_slice(
            acc,
            gathered[s].reshape(sl_batch, seq, _D_MODEL),
            pos_table,
            gamma3,
            beta3,
            s * (sl_batch // bb),
            batch,
        )
    return acc
